# baseline (speedup 1.0000x reference)
import functools

import jax
import jax.numpy as jnp
from jax import lax
from jax.experimental import pallas as pl
from jax.experimental.pallas import tpu as pltpu
from jax.experimental.pallas import tpu_sc as plsc

B, M, N = 64, 32, 8192
K = 16
NC, NS = 2, 16
ROWS_PER_SUBCORE = B // (NC * NS)


def _select_kernel(g_ref, s_ref, eg_ref, vals_ref, idx_ref, sel_ref, s0p_ref):
    x = g_ref[...]
    iota = lax.broadcasted_iota(jnp.int32, (B, N), 1)
    vals = []
    idxs = []
    for _ in range(K):
        m = jnp.max(x, axis=1, keepdims=True)
        idx = jnp.min(jnp.where(x >= m, iota, N), axis=1, keepdims=True)
        vals.append(m)
        idxs.append(idx)
        x = jnp.where(iota == idx, -1.0, x)
    vals_ref[...] = jnp.maximum(jnp.concatenate(vals, axis=1), 1e-8)
    idx_ref[...] = jnp.concatenate(idxs, axis=1)

    sp = s_ref[...]
    sc = sp * eg_ref[...]
    sm = jnp.max(sc, axis=1, keepdims=True)
    iota2 = lax.broadcasted_iota(jnp.int32, (K, N), 1)
    sel = jnp.min(jnp.where(sc >= sm, iota2, N), axis=1)
    sel_ref[0, :] = sel
    s0p_ref[0, :] = jnp.sum(jnp.where(iota2 == sel[:, None], sp, 0.0), axis=1)


def _sc_gather(p128_hbm, sel_hbm, out_hbm, sel_v, idx_v, rows_v, vals_v, sem):
    wid = lax.axis_index("s") * NC + lax.axis_index("c")
    pltpu.sync_copy(sel_hbm, sel_v)
    sel = sel_v[...]
    i16 = lax.iota(jnp.int32, 16)
    for t in range(ROWS_PER_SUBCORE):
        b = wid * ROWS_PER_SUBCORE + t
        flat = b * (M * N) + (K + i16) * N + sel
        lane = jnp.bitwise_and(flat, 127)
        idx_v[...] = lax.shift_right_logical(flat, 7)
        pltpu.async_copy(p128_hbm.at[idx_v], rows_v, sem).wait()
        vals = plsc.load_gather(rows_v, [i16, lane])
        vals_v[...] = jnp.maximum(vals, 1e-8)
        pltpu.sync_copy(vals_v, out_hbm.at[b])


@jax.jit
def kernel(probs):
    eg = jnp.exp(jax.random.gumbel(jax.random.key(42), (K, N), jnp.float32))
    g2 = probs[:, 0, :]
    s2 = probs[0, K:, :]

    vals, idx, sel2d, s0p = pl.pallas_call(
        _select_kernel,
        grid=(),
        in_specs=[
            pl.BlockSpec((B, N), lambda: (0, 0)),
            pl.BlockSpec((K, N), lambda: (0, 0)),
            pl.BlockSpec((K, N), lambda: (0, 0)),
        ],
        out_specs=[
            pl.BlockSpec((B, K), lambda: (0, 0)),
            pl.BlockSpec((B, K), lambda: (0, 0)),
            pl.BlockSpec((1, K), lambda: (0, 0)),
            pl.BlockSpec((1, K), lambda: (0, 0)),
        ],
        out_shape=[
            jax.ShapeDtypeStruct((B, K), jnp.float32),
            jax.ShapeDtypeStruct((B, K), jnp.int32),
            jax.ShapeDtypeStruct((1, K), jnp.int32),
            jax.ShapeDtypeStruct((1, K), jnp.float32),
        ],
    )(g2, s2, eg)

    sel = sel2d[0]

    sc_gather = functools.partial(
        pl.kernel,
        mesh=plsc.VectorSubcoreMesh(core_axis_name="c", subcore_axis_name="s"),
        compiler_params=pltpu.CompilerParams(needs_layout_passes=False),
        out_type=jax.ShapeDtypeStruct((B, K), jnp.float32),
        scratch_types=[
            pltpu.VMEM((K,), jnp.int32),
            pltpu.VMEM((K,), jnp.int32),
            pltpu.VMEM((K, 128), jnp.float32),
            pltpu.VMEM((K,), jnp.float32),
            pltpu.SemaphoreType.DMA,
        ],
    )(_sc_gather)
    sprobs = sc_gather(probs.reshape(B * M * N // 128, 128), sel)

    selected = jnp.concatenate(
        [idx, jnp.broadcast_to(sel[None, :], (B, K))], axis=1)
    prob = jnp.concatenate([vals, sprobs], axis=1)
    return selected, prob
